# baseline (speedup 1.0000x reference)
import functools

import jax
import jax.numpy as jnp
from jax import lax
from jax.experimental import pallas as pl
from jax.experimental.pallas import tpu as pltpu
from jax.experimental.pallas import tpu_sc as plsc

F32 = jnp.float32
I32 = jnp.int32

_N_PAD = 10240
_E_PAD = 327680
_D = 128
_G = 64
_CHUNK = 128
_NCORES = 2
_NSUB = 16
_NW = _NCORES * _NSUB
_EDGES_PER_W = _E_PAD // _NW
_CHUNKS_PER_W = _EDGES_PER_W // _CHUNK
_ROWS_PER_SUB = _N_PAD // _NSUB


def _sc_mesh():
    return plsc.VectorSubcoreMesh(core_axis_name="c", subcore_axis_name="s",
                                  num_cores=_NCORES, num_subcores=_NSUB)


def _prep_body(pack2_hbm, w_hbm, deg_hbm, ed_v, w_v, zero_v, deg_acc, sem):
    cid = lax.axis_index("c")
    sid = lax.axis_index("s")
    wid = cid * _NSUB + sid

    def _z(i, _):
        zero_v[pl.ds(i * 16, 16)] = jnp.zeros((16,), F32)
        return 0
    lax.fori_loop(0, _ROWS_PER_SUB // 16, _z, 0)
    pltpu.sync_copy(zero_v, deg_acc.at[pl.ds(sid * _ROWS_PER_SUB, _ROWS_PER_SUB)])
    plsc.subcore_barrier()

    def _chunk(c, _):
        ci = wid * _CHUNKS_PER_W + c
        pltpu.sync_copy(pack2_hbm.at[ci], ed_v)
        pltpu.sync_copy(w_hbm.at[pl.ds(ci * _CHUNK, _CHUNK)], w_v)
        pltpu.sync_copy(w_v, deg_acc.at[ed_v.at[1]], add=True)
        return 0
    lax.fori_loop(0, _CHUNKS_PER_W, _chunk, 0)

    plsc.subcore_barrier()
    pltpu.sync_copy(deg_acc.at[pl.ds(sid * _ROWS_PER_SUB, _ROWS_PER_SUB)],
                    deg_hbm.at[cid, pl.ds(sid * _ROWS_PER_SUB, _ROWS_PER_SUB)])


def _sc_prep(pack2, w):
    return pl.kernel(
        _prep_body,
        out_type=jax.ShapeDtypeStruct((_NCORES, _N_PAD), F32),
        mesh=_sc_mesh(),
        scratch_types=[
            pltpu.VMEM((2, _CHUNK), I32),
            pltpu.VMEM((_CHUNK,), F32),
            pltpu.VMEM((_ROWS_PER_SUB,), F32),
            pltpu.VMEM_SHARED((_N_PAD,), F32),
            pltpu.SemaphoreType.DMA,
        ],
    )(pack2, w)


_SBLK = 8
_NSBLK = _CHUNKS_PER_W // _SBLK
_NTOT_SBLK = _E_PAD // (_SBLK * _CHUNK)


def _agg_body(src_hbm, dst_hbm, w_hbm, h2_hbm, zeros_hbm, out_hbm,
              src_v, dst_v, w_v, rows_v, acc, sem, ssem):
    cid = lax.axis_index("c")
    sid = lax.axis_index("s")
    wid = cid * _NSUB + sid
    row0 = sid * _ROWS_PER_SUB

    pltpu.sync_copy(zeros_hbm.at[pl.ds(row0, _ROWS_PER_SUB)],
                    acc.at[pl.ds(row0, _ROWS_PER_SUB)])
    plsc.subcore_barrier()

    def _sblk(sb, _):
        si = wid * _NSBLK + sb
        pltpu.sync_copy(src_hbm.at[si], src_v)
        pltpu.sync_copy(dst_hbm.at[si], dst_v)
        pltpu.sync_copy(w_hbm.at[si], w_v)

        @pl.when(sb >= 1)
        def _():
            pltpu.make_async_copy(h2_hbm.at[pl.ds(0, _CHUNK)],
                                  rows_v.at[0], ssem.at[0]).wait()
        pltpu.async_copy(h2_hbm.at[src_v.at[0]], rows_v.at[0], sem.at[0])
        for j in range(_SBLK):
            if j + 1 < _SBLK:
                nsl = (j + 1) % 2
                if j + 1 >= 2:
                    pltpu.make_async_copy(h2_hbm.at[pl.ds(0, _CHUNK)],
                                          rows_v.at[nsl], ssem.at[nsl]).wait()
                else:
                    @pl.when(sb >= 1)
                    def _():
                        pltpu.make_async_copy(h2_hbm.at[pl.ds(0, _CHUNK)],
                                              rows_v.at[nsl], ssem.at[nsl]).wait()
                pltpu.async_copy(h2_hbm.at[src_v.at[j + 1]],
                                 rows_v.at[nsl], sem.at[nsl])
            pltpu.make_async_copy(h2_hbm.at[pl.ds(0, _CHUNK)],
                                  rows_v.at[j % 2], sem.at[j % 2]).wait()

            def _grp(g, _):
                wv = w_v[j, pl.ds(g * 16, 16)]
                for e in range(16):
                    spl = lax.gather(
                        wv, jnp.full((16, 1), e, I32),
                        lax.GatherDimensionNumbers(
                            offset_dims=(), collapsed_slice_dims=(0,),
                            start_index_map=(0,)),
                        slice_sizes=(1,),
                        mode=lax.GatherScatterMode.PROMISE_IN_BOUNDS)
                    r = g * 16 + e
                    for cc in range(8):
                        sl = pl.ds(cc * 16, 16)
                        rows_v[j % 2, r, sl] = rows_v[j % 2, r, sl] * spl
                return 0
            lax.fori_loop(0, _CHUNK // 16, _grp, 0)
            pltpu.async_copy(rows_v.at[j % 2], acc.at[dst_v.at[j]],
                             ssem.at[j % 2], add=True)
        return 0
    lax.fori_loop(0, _NSBLK, _sblk, 0)

    for fsl in (0, 1):
        pltpu.make_async_copy(h2_hbm.at[pl.ds(0, _CHUNK)],
                              rows_v.at[fsl], ssem.at[fsl]).wait()
    plsc.subcore_barrier()
    pltpu.sync_copy(acc.at[pl.ds(row0, _ROWS_PER_SUB)],
                    out_hbm.at[cid, pl.ds(row0, _ROWS_PER_SUB)])


def _sc_agg(srcs, dsts, ws, h2, zeros2d):
    return pl.kernel(
        _agg_body,
        out_type=jax.ShapeDtypeStruct((_NCORES, _N_PAD, _D), F32),
        mesh=_sc_mesh(),
        scratch_types=[
            pltpu.VMEM((_SBLK, _CHUNK), I32),
            pltpu.VMEM((_SBLK, _CHUNK), I32),
            pltpu.VMEM((_SBLK, _CHUNK), F32),
            pltpu.VMEM((2, _CHUNK, _D), F32),
            pltpu.VMEM_SHARED((_N_PAD, _D), F32),
            pltpu.SemaphoreType.DMA((2,)),
            pltpu.SemaphoreType.DMA((2,)),
        ],
    )(srcs, dsts, ws, h2, zeros2d)


def _pre1_body(et_ref, we_ref, be_ref, nt_ref, wl_ref, bl_ref, t_ref, tab2_ref):
    t_ref[...] = jnp.dot(et_ref[...], we_ref[...],
                         preferred_element_type=F32) + be_ref[...]
    tab2_ref[...] = jnp.maximum(
        jnp.dot(nt_ref[...], wl_ref[...], preferred_element_type=F32)
        + bl_ref[...], 0.0)


def _tc_pre1(et_pad, W_edge, b_edge2, nt_pad, W_lin, b_lin2):
    return pl.pallas_call(
        _pre1_body,
        out_shape=(jax.ShapeDtypeStruct((_D, 1), F32),
                   jax.ShapeDtypeStruct((_D, _D), F32)),
    )(et_pad, W_edge, b_edge2, nt_pad, W_lin, b_lin2)


_EBLK = 2048


def _w_body(attr_ref, t_ref, w_ref):
    iota = lax.broadcasted_iota(I32, (_EBLK, _D), 1)
    oh = (iota == attr_ref[...]).astype(F32)
    z = jnp.dot(oh, t_ref[...], preferred_element_type=F32, precision=lax.Precision.HIGHEST)
    w_ref[...] = 1.0 / (1.0 + jnp.exp(-z))


def _tc_w(attr2d, t_tab):
    return pl.pallas_call(
        _w_body,
        grid=(_E_PAD // _EBLK,),
        in_specs=[pl.BlockSpec((_EBLK, 1), lambda i: (i, 0)),
                  pl.BlockSpec((_D, 1), lambda i: (0, 0))],
        out_specs=pl.BlockSpec((_EBLK, 1), lambda i: (i, 0)),
        out_shape=jax.ShapeDtypeStruct((_E_PAD, 1), F32),
    )(attr2d, t_tab)


_BLK = 256
_NBLK = _N_PAD // _BLK


def _pre2_body(x_ref, dA_ref, dB_ref, tab2_ref, w0_ref,
               h0_ref, dinv_ref, h2_ref):
    deg = dA_ref[...] + dB_ref[...]
    dinv = jnp.where(deg > 0, lax.rsqrt(jnp.maximum(deg, 1e-12)), 0.0)
    dinv2d = jnp.broadcast_to(dinv, (_BLK, _D))
    iota = lax.broadcasted_iota(I32, (_BLK, _D), 1)
    oh = (iota == x_ref[...]).astype(F32)
    h0 = jnp.dot(oh, tab2_ref[...], preferred_element_type=F32, precision=lax.Precision.HIGHEST)
    h0_ref[...] = h0
    dinv_ref[...] = dinv2d
    h2_ref[...] = dinv2d * jnp.dot(h0, w0_ref[...], preferred_element_type=F32)


def _tc_pre2(x2d, degA, degB, tab2, W0):
    blk = lambda i: (i, 0)
    fix = lambda i: (0, 0)
    return pl.pallas_call(
        _pre2_body,
        grid=(_NBLK,),
        in_specs=[
            pl.BlockSpec((_BLK, 1), blk),
            pl.BlockSpec((_BLK, 1), blk),
            pl.BlockSpec((_BLK, 1), blk),
            pl.BlockSpec((_D, _D), fix),
            pl.BlockSpec((_D, _D), fix),
        ],
        out_specs=(pl.BlockSpec((_BLK, _D), blk),
                   pl.BlockSpec((_BLK, _D), blk),
                   pl.BlockSpec((_BLK, _D), blk)),
        out_shape=(jax.ShapeDtypeStruct((_N_PAD, _D), F32),
                   jax.ShapeDtypeStruct((_N_PAD, _D), F32),
                   jax.ShapeDtypeStruct((_N_PAD, _D), F32)),
    )(x2d, degA, degB, tab2, W0)


def _ln_block(s, g_ref, be_ref):
    m = jnp.mean(s, axis=1, keepdims=True)
    d = s - m
    v = jnp.mean(d * d, axis=1, keepdims=True)
    return d / jnp.sqrt(v + 1e-5) * g_ref[...] + be_ref[...]


def _layer_mid_body(p0_ref, p1_ref, dinv_ref, hp_ref, b_ref, g_ref, be_ref,
                    wn_ref, hn_ref, h2_ref):
    dinv = dinv_ref[...]
    agg = dinv * (p0_ref[...] + p1_ref[...]) + b_ref[...]
    s = jnp.maximum(agg, 0.0) + hp_ref[...]
    hn = _ln_block(s, g_ref, be_ref)
    hn_ref[...] = hn
    h2_ref[...] = dinv * jnp.dot(hn, wn_ref[...], preferred_element_type=F32)


def _tc_layer_mid(p0, p1, dinv2d, h_prev, b2, g2, be2, Wn):
    blk = lambda i: (i, 0)
    fix = lambda i: (0, 0)
    return pl.pallas_call(
        _layer_mid_body,
        grid=(_NBLK,),
        in_specs=[
            pl.BlockSpec((_BLK, _D), blk),
            pl.BlockSpec((_BLK, _D), blk),
            pl.BlockSpec((_BLK, _D), blk),
            pl.BlockSpec((_BLK, _D), blk),
            pl.BlockSpec((1, _D), fix),
            pl.BlockSpec((1, _D), fix),
            pl.BlockSpec((1, _D), fix),
            pl.BlockSpec((_D, _D), fix),
        ],
        out_specs=(pl.BlockSpec((_BLK, _D), blk),
                   pl.BlockSpec((_BLK, _D), blk)),
        out_shape=(jax.ShapeDtypeStruct((_N_PAD, _D), F32),
                   jax.ShapeDtypeStruct((_N_PAD, _D), F32)),
    )(p0, p1, dinv2d, h_prev, b2, g2, be2, Wn)


def _layer_last_body(p0_ref, p1_ref, dinv_ref, hp_ref, b_ref, g_ref, be_ref,
                     batch_ref, pool_ref):
    agg = dinv_ref[...] * (p0_ref[...] + p1_ref[...]) + b_ref[...]
    s = jnp.maximum(agg, 0.0) + hp_ref[...]
    hn = _ln_block(s, g_ref, be_ref)
    gi = lax.broadcasted_iota(I32, (_G, _BLK), 0)
    oh = (gi == batch_ref[...]).astype(F32)
    part = jnp.dot(oh, hn, preferred_element_type=F32, precision=lax.Precision.HIGHEST)

    @pl.when(pl.program_id(0) == 0)
    def _():
        pool_ref[...] = part

    @pl.when(pl.program_id(0) != 0)
    def _():
        pool_ref[...] = pool_ref[...] + part


def _tc_layer_last(p0, p1, dinv2d, h_prev, b2, g2, be2, batch_row):
    blk = lambda i: (i, 0)
    fix = lambda i: (0, 0)
    return pl.pallas_call(
        _layer_last_body,
        grid=(_NBLK,),
        in_specs=[
            pl.BlockSpec((_BLK, _D), blk),
            pl.BlockSpec((_BLK, _D), blk),
            pl.BlockSpec((_BLK, _D), blk),
            pl.BlockSpec((_BLK, _D), blk),
            pl.BlockSpec((1, _D), fix),
            pl.BlockSpec((1, _D), fix),
            pl.BlockSpec((1, _D), fix),
            pl.BlockSpec((1, _BLK), lambda i: (0, i)),
        ],
        out_specs=pl.BlockSpec((_G, _D), fix),
        out_shape=jax.ShapeDtypeStruct((_G, _D), F32),
    )(p0, p1, dinv2d, h_prev, b2, g2, be2, batch_row)


def _bn_block(y, g_ref, b_ref):
    m = jnp.mean(y, axis=0, keepdims=True)
    d = y - m
    v = jnp.mean(d * d, axis=0, keepdims=True)
    return d / jnp.sqrt(v + 1e-5) * g_ref[...] + b_ref[...]


def _mlp_body(pool_ref, w1_ref, b1_ref, g1_ref, bb1_ref,
              w2_ref, b2_ref, g2_ref, bb2_ref, w3_ref, b3_ref, out_ref):
    y = jnp.maximum(jnp.dot(pool_ref[...], w1_ref[...],
                            preferred_element_type=F32) + b1_ref[...], 0.0)
    y = _bn_block(y, g1_ref, bb1_ref)
    y = jnp.maximum(jnp.dot(y, w2_ref[...],
                            preferred_element_type=F32) + b2_ref[...], 0.0)
    y = _bn_block(y, g2_ref, bb2_ref)
    out_ref[...] = jnp.dot(y, w3_ref[...],
                           preferred_element_type=F32) + b3_ref[...]


def _tc_mlp(pooled, W_fc1, b1, g1, bb1, W_fc2, b2, g2, bb2, W_fc3, b3):
    return pl.pallas_call(
        _mlp_body,
        out_shape=jax.ShapeDtypeStruct((_G, 1), F32),
    )(pooled, W_fc1, b1, g1, bb1, W_fc2, b2, g2, bb2, W_fc3, b3)


@jax.jit
def _run(x, edge_index, edge_attr, batch, node_table, edge_table,
         W_edge, b_edge, W_lin, b_lin,
         gcn_W0, gcn_b0, ln_g0, ln_b0,
         gcn_W1, gcn_b1, ln_g1, ln_b1,
         gcn_W2, gcn_b2, ln_g2, ln_b2,
         W_fc1, b_fc1, bn1_g, bn1_b,
         W_fc2, b_fc2, bn2_g, bn2_b, W_fc3, b_fc3):
    N = x.shape[0]
    E = edge_index.shape[1]
    npad = _N_PAD - N
    epad = _E_PAD - E

    src = edge_index[0].astype(I32)
    dst = edge_index[1].astype(I32)
    pad_i = jnp.arange(epad, dtype=I32)
    src_p = jnp.concatenate([src, pad_i % 128])
    dst_p = jnp.concatenate([dst, N + (pad_i % npad)])
    attr_p = jnp.concatenate([edge_attr.astype(I32), jnp.zeros((epad,), I32)])
    x_p = jnp.concatenate([x.astype(I32), jnp.zeros((npad,), I32)])
    batch_p = jnp.concatenate([batch.astype(I32), jnp.full((npad,), _G, I32)])

    et_pad = jnp.concatenate(
        [edge_table, jnp.zeros((_D - edge_table.shape[0], _D), F32)])
    nt_pad = jnp.concatenate(
        [node_table, jnp.zeros((_D - node_table.shape[0], _D), F32)])

    t_tab, tab2 = _tc_pre1(et_pad, W_edge, b_edge.reshape(1, 1),
                           nt_pad, W_lin, b_lin.reshape(1, _D))
    w_e = _tc_w(attr_p.reshape(_E_PAD, 1), t_tab).reshape(_E_PAD)

    nchunks = _E_PAD // _CHUNK
    pack2 = jnp.stack([src_p.reshape(nchunks, _CHUNK),
                       dst_p.reshape(nchunks, _CHUNK)], axis=1)
    deg_parts = _sc_prep(pack2, w_e)
    src_s = src_p.reshape(_NTOT_SBLK, _SBLK, _CHUNK)
    dst_s = dst_p.reshape(_NTOT_SBLK, _SBLK, _CHUNK)
    w_s = w_e.reshape(_NTOT_SBLK, _SBLK, _CHUNK)
    zeros2d = jnp.zeros((_N_PAD, _D), F32)

    h0, dinv2d, h2 = _tc_pre2(x_p.reshape(_N_PAD, 1),
                              deg_parts[0].reshape(_N_PAD, 1),
                              deg_parts[1].reshape(_N_PAD, 1),
                              tab2, gcn_W0)

    parts = _sc_agg(src_s, dst_s, w_s, h2, zeros2d)
    h1, h2 = _tc_layer_mid(parts[0], parts[1], dinv2d, h0,
                           gcn_b0.reshape(1, _D), ln_g0.reshape(1, _D),
                           ln_b0.reshape(1, _D), gcn_W1)
    parts = _sc_agg(src_s, dst_s, w_s, h2, zeros2d)
    h2r, h2 = _tc_layer_mid(parts[0], parts[1], dinv2d, h1,
                            gcn_b1.reshape(1, _D), ln_g1.reshape(1, _D),
                            ln_b1.reshape(1, _D), gcn_W2)
    parts = _sc_agg(src_s, dst_s, w_s, h2, zeros2d)
    pooled = _tc_layer_last(parts[0], parts[1], dinv2d, h2r,
                            gcn_b2.reshape(1, _D), ln_g2.reshape(1, _D),
                            ln_b2.reshape(1, _D), batch_p.reshape(1, _N_PAD))

    return _tc_mlp(pooled, W_fc1, b_fc1.reshape(1, 64),
                   bn1_g.reshape(1, 64), bn1_b.reshape(1, 64),
                   W_fc2, b_fc2.reshape(1, 32),
                   bn2_g.reshape(1, 32), bn2_b.reshape(1, 32),
                   W_fc3, b_fc3.reshape(1, 1))


def kernel(x, edge_index, edge_attr, batch, size, node_table, edge_table,
           W_edge, b_edge, W_lin, b_lin,
           gcn_W0, gcn_b0, ln_g0, ln_b0,
           gcn_W1, gcn_b1, ln_g1, ln_b1,
           gcn_W2, gcn_b2, ln_g2, ln_b2,
           W_fc1, b_fc1, bn1_g, bn1_b,
           W_fc2, b_fc2, bn2_g, bn2_b, W_fc3, b_fc3):
    del size
    return _run(x, edge_index, edge_attr, batch, node_table, edge_table,
                W_edge, b_edge, W_lin, b_lin,
                gcn_W0, gcn_b0, ln_g0, ln_b0,
                gcn_W1, gcn_b1, ln_g1, ln_b1,
                gcn_W2, gcn_b2, ln_g2, ln_b2,
                W_fc1, b_fc1, bn1_g, bn1_b,
                W_fc2, b_fc2, bn2_g, bn2_b, W_fc3, b_fc3)


# speedup vs baseline: 1.1887x; 1.1887x over previous
import functools

import jax
import jax.numpy as jnp
from jax import lax
from jax.experimental import pallas as pl
from jax.experimental.pallas import tpu as pltpu
from jax.experimental.pallas import tpu_sc as plsc

F32 = jnp.float32
I32 = jnp.int32

_N_PAD = 10240
_E_PAD = 327680
_D = 128
_G = 64
_CHUNK = 128
_NCORES = 2
_NSUB = 16
_NW = _NCORES * _NSUB
_EDGES_PER_W = _E_PAD // _NW
_CHUNKS_PER_W = _EDGES_PER_W // _CHUNK
_ROWS_PER_SUB = _N_PAD // _NSUB


def _sc_mesh():
    return plsc.VectorSubcoreMesh(core_axis_name="c", subcore_axis_name="s",
                                  num_cores=_NCORES, num_subcores=_NSUB)


def _prep_body(dst_hbm, w_hbm, deg_hbm, dst_v, w_v, zero_v, deg_acc):
    cid = lax.axis_index("c")
    sid = lax.axis_index("s")
    wid = cid * _NSUB + sid

    def _z(i, _):
        zero_v[pl.ds(i * 16, 16)] = jnp.zeros((16,), F32)
        return 0
    lax.fori_loop(0, _ROWS_PER_SUB // 16, _z, 0)
    pltpu.sync_copy(zero_v, deg_acc.at[pl.ds(sid * _ROWS_PER_SUB, _ROWS_PER_SUB)])
    plsc.subcore_barrier()

    def _sblk(sb, _):
        si = wid * _NSBLK + sb
        pltpu.sync_copy(dst_hbm.at[si], dst_v)
        pltpu.sync_copy(w_hbm.at[si], w_v)
        for j in range(_SBLK):
            pltpu.sync_copy(w_v.at[j], deg_acc.at[dst_v.at[j]], add=True)
        return 0
    lax.fori_loop(0, _NSBLK, _sblk, 0)

    plsc.subcore_barrier()
    pltpu.sync_copy(deg_acc.at[pl.ds(sid * _ROWS_PER_SUB, _ROWS_PER_SUB)],
                    deg_hbm.at[cid, pl.ds(sid * _ROWS_PER_SUB, _ROWS_PER_SUB)])


def _sc_prep(dsts, ws):
    return pl.kernel(
        _prep_body,
        out_type=jax.ShapeDtypeStruct((_NCORES, _N_PAD), F32),
        mesh=_sc_mesh(),
        scratch_types=[
            pltpu.VMEM((_SBLK, _CHUNK), I32),
            pltpu.VMEM((_SBLK, _CHUNK), F32),
            pltpu.VMEM((_ROWS_PER_SUB,), F32),
            pltpu.VMEM_SHARED((_N_PAD,), F32),
        ],
    )(dsts, ws)


_SBLK = 8
_NSBLK = _CHUNKS_PER_W // _SBLK
_NTOT_SBLK = _E_PAD // (_SBLK * _CHUNK)


def _agg_body(src_hbm, dst_hbm, w_hbm, h2_hbm, zeros_hbm, out_hbm,
              src_v, dst_v, w_v, rows_v, acc, sem):
    cid = lax.axis_index("c")
    sid = lax.axis_index("s")
    wid = cid * _NSUB + sid
    row0 = sid * _ROWS_PER_SUB

    pltpu.sync_copy(zeros_hbm.at[pl.ds(row0, _ROWS_PER_SUB)],
                    acc.at[pl.ds(row0, _ROWS_PER_SUB)])
    plsc.subcore_barrier()

    def _sblk(sb, _):
        si = wid * _NSBLK + sb
        pltpu.sync_copy(src_hbm.at[si], src_v)
        pltpu.sync_copy(dst_hbm.at[si], dst_v)
        pltpu.sync_copy(w_hbm.at[si], w_v)
        pltpu.async_copy(h2_hbm.at[src_v.at[0]], rows_v.at[0], sem.at[0])
        for j in range(_SBLK):
            if j + 1 < _SBLK:
                pltpu.async_copy(h2_hbm.at[src_v.at[j + 1]],
                                 rows_v.at[(j + 1) % 2], sem.at[(j + 1) % 2])
            pltpu.make_async_copy(h2_hbm.at[pl.ds(0, _CHUNK)],
                                  rows_v.at[j % 2], sem.at[j % 2]).wait()

            def _grp(g, _):
                wv = w_v[j, pl.ds(g * 16, 16)]
                for e in range(16):
                    spl = lax.gather(
                        wv, jnp.full((16, 1), e, I32),
                        lax.GatherDimensionNumbers(
                            offset_dims=(), collapsed_slice_dims=(0,),
                            start_index_map=(0,)),
                        slice_sizes=(1,),
                        mode=lax.GatherScatterMode.PROMISE_IN_BOUNDS)
                    r = g * 16 + e
                    for cc in range(8):
                        sl = pl.ds(cc * 16, 16)
                        rows_v[j % 2, r, sl] = rows_v[j % 2, r, sl] * spl
                return 0
            lax.fori_loop(0, _CHUNK // 16, _grp, 0)
            pltpu.sync_copy(rows_v.at[j % 2], acc.at[dst_v.at[j]], add=True)
        return 0
    lax.fori_loop(0, _NSBLK, _sblk, 0)

    plsc.subcore_barrier()
    pltpu.sync_copy(acc.at[pl.ds(row0, _ROWS_PER_SUB)],
                    out_hbm.at[cid, pl.ds(row0, _ROWS_PER_SUB)])


def _sc_agg(srcs, dsts, ws, h2, zeros2d):
    return pl.kernel(
        _agg_body,
        out_type=jax.ShapeDtypeStruct((_NCORES, _N_PAD, _D), F32),
        mesh=_sc_mesh(),
        scratch_types=[
            pltpu.VMEM((_SBLK, _CHUNK), I32),
            pltpu.VMEM((_SBLK, _CHUNK), I32),
            pltpu.VMEM((_SBLK, _CHUNK), F32),
            pltpu.VMEM((2, _CHUNK, _D), F32),
            pltpu.VMEM_SHARED((_N_PAD, _D), F32),
            pltpu.SemaphoreType.DMA((2,)),
        ],
    )(srcs, dsts, ws, h2, zeros2d)


def _pre1w_body(attr_ref, et_ref, we_ref, be_ref, nt_ref, wl_ref, bl_ref,
                w_ref, tab2_ref):
    t = jnp.dot(et_ref[...], we_ref[...],
                preferred_element_type=F32) + be_ref[...]
    iota = lax.broadcasted_iota(I32, (_EBLK, _D), 1)
    oh = (iota == attr_ref[...]).astype(F32)
    z = jnp.dot(oh, t, preferred_element_type=F32,
                precision=lax.Precision.HIGHEST)
    w_ref[...] = 1.0 / (1.0 + jnp.exp(-z))

    @pl.when(pl.program_id(0) == 0)
    def _():
        tab2_ref[...] = jnp.maximum(
            jnp.dot(nt_ref[...], wl_ref[...], preferred_element_type=F32)
            + bl_ref[...], 0.0)


_EBLK = 2048


def _tc_pre1w(attr2d, et_pad, W_edge, b_edge2, nt_pad, W_lin, b_lin2):
    fix = lambda i: (0, 0)
    return pl.pallas_call(
        _pre1w_body,
        grid=(_E_PAD // _EBLK,),
        in_specs=[
            pl.BlockSpec((_EBLK, 1), lambda i: (i, 0)),
            pl.BlockSpec((_D, _D), fix),
            pl.BlockSpec((_D, 1), fix),
            pl.BlockSpec((1, 1), fix),
            pl.BlockSpec((_D, _D), fix),
            pl.BlockSpec((_D, _D), fix),
            pl.BlockSpec((1, _D), fix),
        ],
        out_specs=(pl.BlockSpec((_EBLK, 1), lambda i: (i, 0)),
                   pl.BlockSpec((_D, _D), fix)),
        out_shape=(jax.ShapeDtypeStruct((_E_PAD, 1), F32),
                   jax.ShapeDtypeStruct((_D, _D), F32)),
    )(attr2d, et_pad, W_edge, b_edge2, nt_pad, W_lin, b_lin2)


_BLK = 256
_NBLK = _N_PAD // _BLK


def _pre2_body(x_ref, dA_ref, dB_ref, tab2_ref, w0_ref,
               h0_ref, dinv_ref, h2_ref):
    deg = dA_ref[...] + dB_ref[...]
    dinv = jnp.where(deg > 0, lax.rsqrt(jnp.maximum(deg, 1e-12)), 0.0)
    dinv2d = jnp.broadcast_to(dinv, (_BLK, _D))
    iota = lax.broadcasted_iota(I32, (_BLK, _D), 1)
    oh = (iota == x_ref[...]).astype(F32)
    h0 = jnp.dot(oh, tab2_ref[...], preferred_element_type=F32, precision=lax.Precision.HIGHEST)
    h0_ref[...] = h0
    dinv_ref[...] = dinv2d
    h2_ref[...] = dinv2d * jnp.dot(h0, w0_ref[...], preferred_element_type=F32)


def _tc_pre2(x2d, degA, degB, tab2, W0):
    blk = lambda i: (i, 0)
    fix = lambda i: (0, 0)
    return pl.pallas_call(
        _pre2_body,
        grid=(_NBLK,),
        in_specs=[
            pl.BlockSpec((_BLK, 1), blk),
            pl.BlockSpec((_BLK, 1), blk),
            pl.BlockSpec((_BLK, 1), blk),
            pl.BlockSpec((_D, _D), fix),
            pl.BlockSpec((_D, _D), fix),
        ],
        out_specs=(pl.BlockSpec((_BLK, _D), blk),
                   pl.BlockSpec((_BLK, _D), blk),
                   pl.BlockSpec((_BLK, _D), blk)),
        out_shape=(jax.ShapeDtypeStruct((_N_PAD, _D), F32),
                   jax.ShapeDtypeStruct((_N_PAD, _D), F32),
                   jax.ShapeDtypeStruct((_N_PAD, _D), F32)),
    )(x2d, degA, degB, tab2, W0)


def _ln_block(s, g_ref, be_ref):
    m = jnp.mean(s, axis=1, keepdims=True)
    d = s - m
    v = jnp.mean(d * d, axis=1, keepdims=True)
    return d / jnp.sqrt(v + 1e-5) * g_ref[...] + be_ref[...]


def _layer_mid_body(p0_ref, p1_ref, dinv_ref, hp_ref, b_ref, g_ref, be_ref,
                    wn_ref, hn_ref, h2_ref):
    dinv = dinv_ref[...]
    agg = dinv * (p0_ref[...] + p1_ref[...]) + b_ref[...]
    s = jnp.maximum(agg, 0.0) + hp_ref[...]
    hn = _ln_block(s, g_ref, be_ref)
    hn_ref[...] = hn
    h2_ref[...] = dinv * jnp.dot(hn, wn_ref[...], preferred_element_type=F32)


def _tc_layer_mid(p0, p1, dinv2d, h_prev, b2, g2, be2, Wn):
    blk = lambda i: (i, 0)
    fix = lambda i: (0, 0)
    return pl.pallas_call(
        _layer_mid_body,
        grid=(_NBLK,),
        in_specs=[
            pl.BlockSpec((_BLK, _D), blk),
            pl.BlockSpec((_BLK, _D), blk),
            pl.BlockSpec((_BLK, _D), blk),
            pl.BlockSpec((_BLK, _D), blk),
            pl.BlockSpec((1, _D), fix),
            pl.BlockSpec((1, _D), fix),
            pl.BlockSpec((1, _D), fix),
            pl.BlockSpec((_D, _D), fix),
        ],
        out_specs=(pl.BlockSpec((_BLK, _D), blk),
                   pl.BlockSpec((_BLK, _D), blk)),
        out_shape=(jax.ShapeDtypeStruct((_N_PAD, _D), F32),
                   jax.ShapeDtypeStruct((_N_PAD, _D), F32)),
    )(p0, p1, dinv2d, h_prev, b2, g2, be2, Wn)


def _layer_last_body(p0_ref, p1_ref, dinv_ref, hp_ref, b_ref, g_ref, be_ref,
                     batch_ref, w1_ref, b1_ref, g1_ref, bb1_ref,
                     w2_ref, b2_ref, g2_ref, bb2_ref, w3_ref, b3_ref,
                     out_ref, pool_ref):
    agg = dinv_ref[...] * (p0_ref[...] + p1_ref[...]) + b_ref[...]
    s = jnp.maximum(agg, 0.0) + hp_ref[...]
    hn = _ln_block(s, g_ref, be_ref)
    gi = lax.broadcasted_iota(I32, (_G, _BLK), 0)
    oh = (gi == batch_ref[...]).astype(F32)
    part = jnp.dot(oh, hn, preferred_element_type=F32,
                   precision=lax.Precision.HIGHEST)

    @pl.when(pl.program_id(0) == 0)
    def _():
        pool_ref[...] = part

    @pl.when(pl.program_id(0) != 0)
    def _():
        pool_ref[...] = pool_ref[...] + part

    @pl.when(pl.program_id(0) == _NBLK - 1)
    def _():
        y = jnp.maximum(jnp.dot(pool_ref[...], w1_ref[...],
                                preferred_element_type=F32) + b1_ref[...], 0.0)
        y = _bn_block(y, g1_ref, bb1_ref)
        y = jnp.maximum(jnp.dot(y, w2_ref[...],
                                preferred_element_type=F32) + b2_ref[...], 0.0)
        y = _bn_block(y, g2_ref, bb2_ref)
        out_ref[...] = jnp.dot(y, w3_ref[...],
                               preferred_element_type=F32) + b3_ref[...]


def _tc_layer_last(p0, p1, dinv2d, h_prev, b2, g2, be2, batch_row,
                   W_fc1, b1, g1, bb1, W_fc2, bb2w, g2b, bb2b, W_fc3, b3):
    blk = lambda i: (i, 0)
    fix = lambda i: (0, 0)
    return pl.pallas_call(
        _layer_last_body,
        grid=(_NBLK,),
        in_specs=[
            pl.BlockSpec((_BLK, _D), blk),
            pl.BlockSpec((_BLK, _D), blk),
            pl.BlockSpec((_BLK, _D), blk),
            pl.BlockSpec((_BLK, _D), blk),
            pl.BlockSpec((1, _D), fix),
            pl.BlockSpec((1, _D), fix),
            pl.BlockSpec((1, _D), fix),
            pl.BlockSpec((1, _BLK), lambda i: (0, i)),
            pl.BlockSpec((_D, 64), fix),
            pl.BlockSpec((1, 64), fix),
            pl.BlockSpec((1, 64), fix),
            pl.BlockSpec((1, 64), fix),
            pl.BlockSpec((64, 32), fix),
            pl.BlockSpec((1, 32), fix),
            pl.BlockSpec((1, 32), fix),
            pl.BlockSpec((1, 32), fix),
            pl.BlockSpec((32, 1), fix),
            pl.BlockSpec((1, 1), fix),
        ],
        out_specs=pl.BlockSpec((_G, 1), fix),
        out_shape=jax.ShapeDtypeStruct((_G, 1), F32),
        scratch_shapes=[pltpu.VMEM((_G, _D), F32)],
    )(p0, p1, dinv2d, h_prev, b2, g2, be2, batch_row,
      W_fc1, b1, g1, bb1, W_fc2, bb2w, g2b, bb2b, W_fc3, b3)


def _bn_block(y, g_ref, b_ref):
    m = jnp.mean(y, axis=0, keepdims=True)
    d = y - m
    v = jnp.mean(d * d, axis=0, keepdims=True)
    return d / jnp.sqrt(v + 1e-5) * g_ref[...] + b_ref[...]


@jax.jit
def _run(x, edge_index, edge_attr, batch, node_table, edge_table,
         W_edge, b_edge, W_lin, b_lin,
         gcn_W0, gcn_b0, ln_g0, ln_b0,
         gcn_W1, gcn_b1, ln_g1, ln_b1,
         gcn_W2, gcn_b2, ln_g2, ln_b2,
         W_fc1, b_fc1, bn1_g, bn1_b,
         W_fc2, b_fc2, bn2_g, bn2_b, W_fc3, b_fc3):
    N = x.shape[0]
    E = edge_index.shape[1]
    npad = _N_PAD - N
    epad = _E_PAD - E

    src = edge_index[0].astype(I32)
    dst = edge_index[1].astype(I32)
    pad_i = jnp.arange(epad, dtype=I32)
    src_p = jnp.concatenate([src, pad_i % 128])
    dst_p = jnp.concatenate([dst, N + (pad_i % npad)])
    attr_p = jnp.concatenate([edge_attr.astype(I32), jnp.zeros((epad,), I32)])
    x_p = jnp.concatenate([x.astype(I32), jnp.zeros((npad,), I32)])
    batch_p = jnp.concatenate([batch.astype(I32), jnp.full((npad,), _G, I32)])

    et_pad = jnp.concatenate(
        [edge_table, jnp.zeros((_D - edge_table.shape[0], _D), F32)])
    nt_pad = jnp.concatenate(
        [node_table, jnp.zeros((_D - node_table.shape[0], _D), F32)])

    w2d, tab2 = _tc_pre1w(attr_p.reshape(_E_PAD, 1), et_pad, W_edge,
                          b_edge.reshape(1, 1), nt_pad, W_lin,
                          b_lin.reshape(1, _D))
    src_s = src_p.reshape(_NTOT_SBLK, _SBLK, _CHUNK)
    dst_s = dst_p.reshape(_NTOT_SBLK, _SBLK, _CHUNK)
    w_s = w2d.reshape(_NTOT_SBLK, _SBLK, _CHUNK)
    deg_parts = _sc_prep(dst_s, w_s)
    zeros2d = jnp.zeros((_N_PAD, _D), F32)

    h0, dinv2d, h2 = _tc_pre2(x_p.reshape(_N_PAD, 1),
                              deg_parts[0].reshape(_N_PAD, 1),
                              deg_parts[1].reshape(_N_PAD, 1),
                              tab2, gcn_W0)

    parts = _sc_agg(src_s, dst_s, w_s, h2, zeros2d)
    h1, h2 = _tc_layer_mid(parts[0], parts[1], dinv2d, h0,
                           gcn_b0.reshape(1, _D), ln_g0.reshape(1, _D),
                           ln_b0.reshape(1, _D), gcn_W1)
    parts = _sc_agg(src_s, dst_s, w_s, h2, zeros2d)
    h2r, h2 = _tc_layer_mid(parts[0], parts[1], dinv2d, h1,
                            gcn_b1.reshape(1, _D), ln_g1.reshape(1, _D),
                            ln_b1.reshape(1, _D), gcn_W2)
    parts = _sc_agg(src_s, dst_s, w_s, h2, zeros2d)
    return _tc_layer_last(parts[0], parts[1], dinv2d, h2r,
                          gcn_b2.reshape(1, _D), ln_g2.reshape(1, _D),
                          ln_b2.reshape(1, _D), batch_p.reshape(1, _N_PAD),
                          W_fc1, b_fc1.reshape(1, 64),
                          bn1_g.reshape(1, 64), bn1_b.reshape(1, 64),
                          W_fc2, b_fc2.reshape(1, 32),
                          bn2_g.reshape(1, 32), bn2_b.reshape(1, 32),
                          W_fc3, b_fc3.reshape(1, 1))


def kernel(x, edge_index, edge_attr, batch, size, node_table, edge_table,
           W_edge, b_edge, W_lin, b_lin,
           gcn_W0, gcn_b0, ln_g0, ln_b0,
           gcn_W1, gcn_b1, ln_g1, ln_b1,
           gcn_W2, gcn_b2, ln_g2, ln_b2,
           W_fc1, b_fc1, bn1_g, bn1_b,
           W_fc2, b_fc2, bn2_g, bn2_b, W_fc3, b_fc3):
    del size
    return _run(x, edge_index, edge_attr, batch, node_table, edge_table,
                W_edge, b_edge, W_lin, b_lin,
                gcn_W0, gcn_b0, ln_g0, ln_b0,
                gcn_W1, gcn_b1, ln_g1, ln_b1,
                gcn_W2, gcn_b2, ln_g2, ln_b2,
                W_fc1, b_fc1, bn1_g, bn1_b,
                W_fc2, b_fc2, bn2_g, bn2_b, W_fc3, b_fc3)


# speedup vs baseline: 1.2452x; 1.0475x over previous
import functools

import jax
import jax.numpy as jnp
from jax import lax
from jax.experimental import pallas as pl
from jax.experimental.pallas import tpu as pltpu
from jax.experimental.pallas import tpu_sc as plsc

F32 = jnp.float32
I32 = jnp.int32

_N_PAD = 10240
_E_PAD = 327680
_D = 128
_G = 64
_CHUNK = 128
_NCORES = 2
_NSUB = 16
_NW = _NCORES * _NSUB
_EDGES_PER_W = _E_PAD // _NW
_CHUNKS_PER_W = _EDGES_PER_W // _CHUNK
_ROWS_PER_SUB = _N_PAD // _NSUB


def _sc_mesh():
    return plsc.VectorSubcoreMesh(core_axis_name="c", subcore_axis_name="s",
                                  num_cores=_NCORES, num_subcores=_NSUB)


def _prep_body(dst_hbm, w_hbm, deg_hbm, dst_v, w_v, zero_v, deg_acc):
    cid = lax.axis_index("c")
    sid = lax.axis_index("s")
    wid = cid * _NSUB + sid

    def _z(i, _):
        zero_v[pl.ds(i * 16, 16)] = jnp.zeros((16,), F32)
        return 0
    lax.fori_loop(0, _ROWS_PER_SUB // 16, _z, 0)
    pltpu.sync_copy(zero_v, deg_acc.at[pl.ds(sid * _ROWS_PER_SUB, _ROWS_PER_SUB)])
    plsc.subcore_barrier()

    def _sblk(sb, _):
        si = wid * _NSBLK + sb
        pltpu.sync_copy(dst_hbm.at[si], dst_v)
        pltpu.sync_copy(w_hbm.at[si], w_v)
        for j in range(_SBLK):
            pltpu.sync_copy(w_v.at[j], deg_acc.at[dst_v.at[j]], add=True)
        return 0
    lax.fori_loop(0, _NSBLK, _sblk, 0)

    plsc.subcore_barrier()
    pltpu.sync_copy(deg_acc.at[pl.ds(sid * _ROWS_PER_SUB, _ROWS_PER_SUB)],
                    deg_hbm.at[cid, pl.ds(sid * _ROWS_PER_SUB, _ROWS_PER_SUB)])


def _sc_prep(dsts, ws):
    return pl.kernel(
        _prep_body,
        out_type=jax.ShapeDtypeStruct((_NCORES, _N_PAD), F32),
        mesh=_sc_mesh(),
        scratch_types=[
            pltpu.VMEM((_SBLK, _CHUNK), I32),
            pltpu.VMEM((_SBLK, _CHUNK), F32),
            pltpu.VMEM((_ROWS_PER_SUB,), F32),
            pltpu.VMEM_SHARED((_N_PAD,), F32),
        ],
    )(dsts, ws)


_SBLK = 8
_NSBLK = _CHUNKS_PER_W // _SBLK
_NTOT_SBLK = _E_PAD // (_SBLK * _CHUNK)


def _agg_body(src_hbm, dst_hbm, w_hbm, h2_hbm, zeros_hbm, out_hbm,
              src_v, dst_v, w_v, rows_v, acc, sem):
    cid = lax.axis_index("c")
    sid = lax.axis_index("s")
    wid = cid * _NSUB + sid
    row0 = sid * _ROWS_PER_SUB

    pltpu.sync_copy(zeros_hbm.at[pl.ds(row0, _ROWS_PER_SUB)],
                    acc.at[pl.ds(row0, _ROWS_PER_SUB)])
    plsc.subcore_barrier()

    def _sblk(sb, _):
        si = wid * _NSBLK + sb
        pltpu.sync_copy(src_hbm.at[si], src_v)
        pltpu.sync_copy(dst_hbm.at[si], dst_v)
        pltpu.sync_copy(w_hbm.at[si], w_v)
        pltpu.async_copy(h2_hbm.at[src_v.at[0]], rows_v.at[0], sem.at[0])
        for j in range(_SBLK):
            if j + 1 < _SBLK:
                pltpu.async_copy(h2_hbm.at[src_v.at[j + 1]],
                                 rows_v.at[(j + 1) % 2], sem.at[(j + 1) % 2])
            pltpu.make_async_copy(h2_hbm.at[pl.ds(0, _CHUNK)],
                                  rows_v.at[j % 2], sem.at[j % 2]).wait()

            def _grp(g, _):
                wv = w_v[j, pl.ds(g * 16, 16)]
                for e in range(16):
                    spl = lax.gather(
                        wv, jnp.full((16, 1), e, I32),
                        lax.GatherDimensionNumbers(
                            offset_dims=(), collapsed_slice_dims=(0,),
                            start_index_map=(0,)),
                        slice_sizes=(1,),
                        mode=lax.GatherScatterMode.PROMISE_IN_BOUNDS)
                    r = g * 16 + e
                    for cc in range(8):
                        sl = pl.ds(cc * 16, 16)
                        rows_v[j % 2, r, sl] = rows_v[j % 2, r, sl] * spl
                return 0
            lax.fori_loop(0, _CHUNK // 16, _grp, 0)
            pltpu.sync_copy(rows_v.at[j % 2], acc.at[dst_v.at[j]], add=True)
        return 0
    lax.fori_loop(0, _NSBLK, _sblk, 0)

    plsc.subcore_barrier()
    pltpu.sync_copy(acc.at[pl.ds(row0, _ROWS_PER_SUB)],
                    out_hbm.at[cid, pl.ds(row0, _ROWS_PER_SUB)])


def _sc_agg(srcs, dsts, ws, h2, zeros2d):
    return pl.kernel(
        _agg_body,
        out_type=jax.ShapeDtypeStruct((_NCORES, _N_PAD, _D), F32),
        mesh=_sc_mesh(),
        scratch_types=[
            pltpu.VMEM((_SBLK, _CHUNK), I32),
            pltpu.VMEM((_SBLK, _CHUNK), I32),
            pltpu.VMEM((_SBLK, _CHUNK), F32),
            pltpu.VMEM((2, _CHUNK, _D), F32),
            pltpu.VMEM_SHARED((_N_PAD, _D), F32),
            pltpu.SemaphoreType.DMA((2,)),
        ],
    )(srcs, dsts, ws, h2, zeros2d)


def _pre1w_body(attr_ref, et_ref, we_ref, be_ref, nt_ref, wl_ref, bl_ref,
                w_ref, tab2_ref):
    t = jnp.dot(et_ref[...], we_ref[...],
                preferred_element_type=F32) + be_ref[...]
    iota = lax.broadcasted_iota(I32, (_EBLK, _D), 1)
    oh = (iota == attr_ref[...]).astype(F32)
    z = jnp.dot(oh, t, preferred_element_type=F32,
                precision=lax.Precision.HIGHEST)
    w_ref[...] = 1.0 / (1.0 + jnp.exp(-z))

    @pl.when(pl.program_id(0) == 0)
    def _():
        tab2_ref[...] = jnp.maximum(
            jnp.dot(nt_ref[...], wl_ref[...], preferred_element_type=F32)
            + bl_ref[...], 0.0)


_EBLK = 4096


def _tc_pre1w(attr2d, et_pad, W_edge, b_edge2, nt_pad, W_lin, b_lin2):
    fix = lambda i: (0, 0)
    return pl.pallas_call(
        _pre1w_body,
        grid=(_E_PAD // _EBLK,),
        in_specs=[
            pl.BlockSpec((_EBLK, 1), lambda i: (i, 0)),
            pl.BlockSpec((_D, _D), fix),
            pl.BlockSpec((_D, 1), fix),
            pl.BlockSpec((1, 1), fix),
            pl.BlockSpec((_D, _D), fix),
            pl.BlockSpec((_D, _D), fix),
            pl.BlockSpec((1, _D), fix),
        ],
        out_specs=(pl.BlockSpec((_EBLK, 1), lambda i: (i, 0)),
                   pl.BlockSpec((_D, _D), fix)),
        out_shape=(jax.ShapeDtypeStruct((_E_PAD, 1), F32),
                   jax.ShapeDtypeStruct((_D, _D), F32)),
    )(attr2d, et_pad, W_edge, b_edge2, nt_pad, W_lin, b_lin2)


_BLK = 256
_NBLK = _N_PAD // _BLK


def _pre2_body(x_ref, dA_ref, dB_ref, tab2_ref, w0_ref,
               h0_ref, dinv_ref, h2_ref):
    deg = dA_ref[...] + dB_ref[...]
    dinv = jnp.where(deg > 0, lax.rsqrt(jnp.maximum(deg, 1e-12)), 0.0)
    dinv2d = jnp.broadcast_to(dinv, (_BLK, _D))
    iota = lax.broadcasted_iota(I32, (_BLK, _D), 1)
    oh = (iota == x_ref[...]).astype(F32)
    h0 = jnp.dot(oh, tab2_ref[...], preferred_element_type=F32, precision=lax.Precision.HIGHEST)
    h0_ref[...] = h0
    dinv_ref[...] = dinv2d
    h2_ref[...] = dinv2d * jnp.dot(h0, w0_ref[...], preferred_element_type=F32)


def _tc_pre2(x2d, degA, degB, tab2, W0):
    blk = lambda i: (i, 0)
    fix = lambda i: (0, 0)
    return pl.pallas_call(
        _pre2_body,
        grid=(_NBLK,),
        in_specs=[
            pl.BlockSpec((_BLK, 1), blk),
            pl.BlockSpec((_BLK, 1), blk),
            pl.BlockSpec((_BLK, 1), blk),
            pl.BlockSpec((_D, _D), fix),
            pl.BlockSpec((_D, _D), fix),
        ],
        out_specs=(pl.BlockSpec((_BLK, _D), blk),
                   pl.BlockSpec((_BLK, _D), blk),
                   pl.BlockSpec((_BLK, _D), blk)),
        out_shape=(jax.ShapeDtypeStruct((_N_PAD, _D), F32),
                   jax.ShapeDtypeStruct((_N_PAD, _D), F32),
                   jax.ShapeDtypeStruct((_N_PAD, _D), F32)),
    )(x2d, degA, degB, tab2, W0)


def _ln_block(s, g_ref, be_ref):
    m = jnp.mean(s, axis=1, keepdims=True)
    d = s - m
    v = jnp.mean(d * d, axis=1, keepdims=True)
    return d / jnp.sqrt(v + 1e-5) * g_ref[...] + be_ref[...]


def _layer_mid_body(p0_ref, p1_ref, dinv_ref, hp_ref, b_ref, g_ref, be_ref,
                    wn_ref, hn_ref, h2_ref):
    dinv = dinv_ref[...]
    agg = dinv * (p0_ref[...] + p1_ref[...]) + b_ref[...]
    s = jnp.maximum(agg, 0.0) + hp_ref[...]
    hn = _ln_block(s, g_ref, be_ref)
    hn_ref[...] = hn
    h2_ref[...] = dinv * jnp.dot(hn, wn_ref[...], preferred_element_type=F32)


def _tc_layer_mid(p0, p1, dinv2d, h_prev, b2, g2, be2, Wn):
    blk = lambda i: (i, 0)
    fix = lambda i: (0, 0)
    return pl.pallas_call(
        _layer_mid_body,
        grid=(_NBLK,),
        in_specs=[
            pl.BlockSpec((_BLK, _D), blk),
            pl.BlockSpec((_BLK, _D), blk),
            pl.BlockSpec((_BLK, _D), blk),
            pl.BlockSpec((_BLK, _D), blk),
            pl.BlockSpec((1, _D), fix),
            pl.BlockSpec((1, _D), fix),
            pl.BlockSpec((1, _D), fix),
            pl.BlockSpec((_D, _D), fix),
        ],
        out_specs=(pl.BlockSpec((_BLK, _D), blk),
                   pl.BlockSpec((_BLK, _D), blk)),
        out_shape=(jax.ShapeDtypeStruct((_N_PAD, _D), F32),
                   jax.ShapeDtypeStruct((_N_PAD, _D), F32)),
    )(p0, p1, dinv2d, h_prev, b2, g2, be2, Wn)


def _layer_last_body(p0_ref, p1_ref, dinv_ref, hp_ref, b_ref, g_ref, be_ref,
                     batch_ref, w1_ref, b1_ref, g1_ref, bb1_ref,
                     w2_ref, b2_ref, g2_ref, bb2_ref, w3_ref, b3_ref,
                     out_ref, pool_ref):
    agg = dinv_ref[...] * (p0_ref[...] + p1_ref[...]) + b_ref[...]
    s = jnp.maximum(agg, 0.0) + hp_ref[...]
    hn = _ln_block(s, g_ref, be_ref)
    gi = lax.broadcasted_iota(I32, (_G, _BLK), 0)
    oh = (gi == batch_ref[...]).astype(F32)
    part = jnp.dot(oh, hn, preferred_element_type=F32,
                   precision=lax.Precision.HIGHEST)

    @pl.when(pl.program_id(0) == 0)
    def _():
        pool_ref[...] = part

    @pl.when(pl.program_id(0) != 0)
    def _():
        pool_ref[...] = pool_ref[...] + part

    @pl.when(pl.program_id(0) == _NBLK - 1)
    def _():
        y = jnp.maximum(jnp.dot(pool_ref[...], w1_ref[...],
                                preferred_element_type=F32) + b1_ref[...], 0.0)
        y = _bn_block(y, g1_ref, bb1_ref)
        y = jnp.maximum(jnp.dot(y, w2_ref[...],
                                preferred_element_type=F32) + b2_ref[...], 0.0)
        y = _bn_block(y, g2_ref, bb2_ref)
        out_ref[...] = jnp.dot(y, w3_ref[...],
                               preferred_element_type=F32) + b3_ref[...]


def _tc_layer_last(p0, p1, dinv2d, h_prev, b2, g2, be2, batch_row,
                   W_fc1, b1, g1, bb1, W_fc2, bb2w, g2b, bb2b, W_fc3, b3):
    blk = lambda i: (i, 0)
    fix = lambda i: (0, 0)
    return pl.pallas_call(
        _layer_last_body,
        grid=(_NBLK,),
        in_specs=[
            pl.BlockSpec((_BLK, _D), blk),
            pl.BlockSpec((_BLK, _D), blk),
            pl.BlockSpec((_BLK, _D), blk),
            pl.BlockSpec((_BLK, _D), blk),
            pl.BlockSpec((1, _D), fix),
            pl.BlockSpec((1, _D), fix),
            pl.BlockSpec((1, _D), fix),
            pl.BlockSpec((1, _BLK), lambda i: (0, i)),
            pl.BlockSpec((_D, 64), fix),
            pl.BlockSpec((1, 64), fix),
            pl.BlockSpec((1, 64), fix),
            pl.BlockSpec((1, 64), fix),
            pl.BlockSpec((64, 32), fix),
            pl.BlockSpec((1, 32), fix),
            pl.BlockSpec((1, 32), fix),
            pl.BlockSpec((1, 32), fix),
            pl.BlockSpec((32, 1), fix),
            pl.BlockSpec((1, 1), fix),
        ],
        out_specs=pl.BlockSpec((_G, 1), fix),
        out_shape=jax.ShapeDtypeStruct((_G, 1), F32),
        scratch_shapes=[pltpu.VMEM((_G, _D), F32)],
    )(p0, p1, dinv2d, h_prev, b2, g2, be2, batch_row,
      W_fc1, b1, g1, bb1, W_fc2, bb2w, g2b, bb2b, W_fc3, b3)


def _bn_block(y, g_ref, b_ref):
    m = jnp.mean(y, axis=0, keepdims=True)
    d = y - m
    v = jnp.mean(d * d, axis=0, keepdims=True)
    return d / jnp.sqrt(v + 1e-5) * g_ref[...] + b_ref[...]


@jax.jit
def _run(x, edge_index, edge_attr, batch, node_table, edge_table,
         W_edge, b_edge, W_lin, b_lin,
         gcn_W0, gcn_b0, ln_g0, ln_b0,
         gcn_W1, gcn_b1, ln_g1, ln_b1,
         gcn_W2, gcn_b2, ln_g2, ln_b2,
         W_fc1, b_fc1, bn1_g, bn1_b,
         W_fc2, b_fc2, bn2_g, bn2_b, W_fc3, b_fc3):
    N = x.shape[0]
    E = edge_index.shape[1]
    npad = _N_PAD - N
    epad = _E_PAD - E

    pad_i = jnp.arange(epad, dtype=I32)
    epack = jnp.concatenate(
        [jnp.stack([edge_index[0].astype(I32),
                    edge_index[1].astype(I32),
                    edge_attr.astype(I32)], axis=0),
         jnp.stack([pad_i % 128, N + (pad_i % npad),
                    jnp.zeros((epad,), I32)], axis=0)], axis=1)
    src_p = epack[0]
    dst_p = epack[1]
    attr_p = epack[2]
    x_p = jnp.concatenate([x.astype(I32), jnp.zeros((npad,), I32)])
    batch_p = jnp.concatenate([batch.astype(I32), jnp.full((npad,), _G, I32)])

    et_pad = jnp.concatenate(
        [edge_table, jnp.zeros((_D - edge_table.shape[0], _D), F32)])
    nt_pad = jnp.concatenate(
        [node_table, jnp.zeros((_D - node_table.shape[0], _D), F32)])

    w2d, tab2 = _tc_pre1w(attr_p.reshape(_E_PAD, 1), et_pad, W_edge,
                          b_edge.reshape(1, 1), nt_pad, W_lin,
                          b_lin.reshape(1, _D))
    src_s = src_p.reshape(_NTOT_SBLK, _SBLK, _CHUNK)
    dst_s = dst_p.reshape(_NTOT_SBLK, _SBLK, _CHUNK)
    w_s = w2d.reshape(_NTOT_SBLK, _SBLK, _CHUNK)
    deg_parts = _sc_prep(dst_s, w_s)
    zeros2d = jnp.zeros((_N_PAD, _D), F32)

    h0, dinv2d, h2 = _tc_pre2(x_p.reshape(_N_PAD, 1),
                              deg_parts[0].reshape(_N_PAD, 1),
                              deg_parts[1].reshape(_N_PAD, 1),
                              tab2, gcn_W0)

    parts = _sc_agg(src_s, dst_s, w_s, h2, zeros2d)
    h1, h2 = _tc_layer_mid(parts[0], parts[1], dinv2d, h0,
                           gcn_b0.reshape(1, _D), ln_g0.reshape(1, _D),
                           ln_b0.reshape(1, _D), gcn_W1)
    parts = _sc_agg(src_s, dst_s, w_s, h2, zeros2d)
    h2r, h2 = _tc_layer_mid(parts[0], parts[1], dinv2d, h1,
                            gcn_b1.reshape(1, _D), ln_g1.reshape(1, _D),
                            ln_b1.reshape(1, _D), gcn_W2)
    parts = _sc_agg(src_s, dst_s, w_s, h2, zeros2d)
    return _tc_layer_last(parts[0], parts[1], dinv2d, h2r,
                          gcn_b2.reshape(1, _D), ln_g2.reshape(1, _D),
                          ln_b2.reshape(1, _D), batch_p.reshape(1, _N_PAD),
                          W_fc1, b_fc1.reshape(1, 64),
                          bn1_g.reshape(1, 64), bn1_b.reshape(1, 64),
                          W_fc2, b_fc2.reshape(1, 32),
                          bn2_g.reshape(1, 32), bn2_b.reshape(1, 32),
                          W_fc3, b_fc3.reshape(1, 1))


def kernel(x, edge_index, edge_attr, batch, size, node_table, edge_table,
           W_edge, b_edge, W_lin, b_lin,
           gcn_W0, gcn_b0, ln_g0, ln_b0,
           gcn_W1, gcn_b1, ln_g1, ln_b1,
           gcn_W2, gcn_b2, ln_g2, ln_b2,
           W_fc1, b_fc1, bn1_g, bn1_b,
           W_fc2, b_fc2, bn2_g, bn2_b, W_fc3, b_fc3):
    del size
    return _run(x, edge_index, edge_attr, batch, node_table, edge_table,
                W_edge, b_edge, W_lin, b_lin,
                gcn_W0, gcn_b0, ln_g0, ln_b0,
                gcn_W1, gcn_b1, ln_g1, ln_b1,
                gcn_W2, gcn_b2, ln_g2, ln_b2,
                W_fc1, b_fc1, bn1_g, bn1_b,
                W_fc2, b_fc2, bn2_g, bn2_b, W_fc3, b_fc3)


# speedup vs baseline: 1.3363x; 1.0732x over previous
import functools

import jax
import jax.numpy as jnp
from jax import lax
from jax.experimental import pallas as pl
from jax.experimental.pallas import tpu as pltpu
from jax.experimental.pallas import tpu_sc as plsc

F32 = jnp.float32
I32 = jnp.int32

_N_PAD = 10240
_E_PAD = 327680
_D = 128
_G = 64
_CHUNK = 128
_NCORES = 2
_NSUB = 16
_NW = _NCORES * _NSUB
_EDGES_PER_W = _E_PAD // _NW
_CHUNKS_PER_W = _EDGES_PER_W // _CHUNK
_ROWS_PER_SUB = _N_PAD // _NSUB


def _sc_mesh():
    return plsc.VectorSubcoreMesh(core_axis_name="c", subcore_axis_name="s",
                                  num_cores=_NCORES, num_subcores=_NSUB)


def _prep_body(dst_hbm, w_hbm, deg_hbm, dst_v, w_v, zero_v, deg_acc):
    cid = lax.axis_index("c")
    sid = lax.axis_index("s")
    wid = cid * _NSUB + sid

    def _z(i, _):
        zero_v[pl.ds(i * 16, 16)] = jnp.zeros((16,), F32)
        return 0
    lax.fori_loop(0, _ROWS_PER_SUB // 16, _z, 0)
    pltpu.sync_copy(zero_v, deg_acc.at[pl.ds(sid * _ROWS_PER_SUB, _ROWS_PER_SUB)])
    plsc.subcore_barrier()

    def _sblk(sb, _):
        si = wid * _NSBLK + sb
        pltpu.sync_copy(dst_hbm.at[si], dst_v)
        pltpu.sync_copy(w_hbm.at[si], w_v)
        for j in range(_SBLK):
            pltpu.sync_copy(w_v.at[j], deg_acc.at[dst_v.at[j]], add=True)
        return 0
    lax.fori_loop(0, _NSBLK, _sblk, 0)

    plsc.subcore_barrier()
    pltpu.sync_copy(deg_acc.at[pl.ds(sid * _ROWS_PER_SUB, _ROWS_PER_SUB)],
                    deg_hbm.at[cid, pl.ds(sid * _ROWS_PER_SUB, _ROWS_PER_SUB)])


def _sc_prep(dsts, ws):
    return pl.kernel(
        _prep_body,
        out_type=jax.ShapeDtypeStruct((_NCORES, _N_PAD), F32),
        mesh=_sc_mesh(),
        scratch_types=[
            pltpu.VMEM((_SBLK, _CHUNK), I32),
            pltpu.VMEM((_SBLK, _CHUNK), F32),
            pltpu.VMEM((_ROWS_PER_SUB,), F32),
            pltpu.VMEM_SHARED((_N_PAD,), F32),
        ],
    )(dsts, ws)


_SBLK = 8
_NSBLK = _CHUNKS_PER_W // _SBLK
_NTOT_SBLK = _E_PAD // (_SBLK * _CHUNK)


def _agg_body(src_hbm, dst_hbm, w_hbm, h2_hbm, zeros_hbm, out_hbm,
              src_v, dst_v, w_v, rows_v, acc, sem):
    cid = lax.axis_index("c")
    sid = lax.axis_index("s")
    wid = cid * _NSUB + sid
    row0 = sid * _ROWS_PER_SUB

    pltpu.sync_copy(zeros_hbm.at[pl.ds(row0, _ROWS_PER_SUB)],
                    acc.at[pl.ds(row0, _ROWS_PER_SUB)])
    plsc.subcore_barrier()

    def _sblk(sb, _):
        si = wid * _NSBLK + sb
        pltpu.sync_copy(src_hbm.at[si], src_v)
        pltpu.sync_copy(dst_hbm.at[si], dst_v)
        pltpu.sync_copy(w_hbm.at[si], w_v)
        pltpu.async_copy(h2_hbm.at[src_v.at[0]], rows_v.at[0], sem.at[0])
        for j in range(_SBLK):
            if j + 1 < _SBLK:
                pltpu.async_copy(h2_hbm.at[src_v.at[j + 1]],
                                 rows_v.at[(j + 1) % 2], sem.at[(j + 1) % 2])
            pltpu.make_async_copy(h2_hbm.at[pl.ds(0, _CHUNK)],
                                  rows_v.at[j % 2], sem.at[j % 2]).wait()

            def _grp(g, _):
                wv = w_v[j, pl.ds(g * 16, 16)]
                for e in range(16):
                    spl = lax.gather(
                        wv, jnp.full((16, 1), e, I32),
                        lax.GatherDimensionNumbers(
                            offset_dims=(), collapsed_slice_dims=(0,),
                            start_index_map=(0,)),
                        slice_sizes=(1,),
                        mode=lax.GatherScatterMode.PROMISE_IN_BOUNDS)
                    r = g * 16 + e
                    for cc in range(8):
                        sl = pl.ds(cc * 16, 16)
                        rows_v[j % 2, r, sl] = rows_v[j % 2, r, sl] * spl
                return 0
            lax.fori_loop(0, _CHUNK // 16, _grp, 0)
            pltpu.sync_copy(rows_v.at[j % 2], acc.at[dst_v.at[j]], add=True)
        return 0
    lax.fori_loop(0, _NSBLK, _sblk, 0)

    plsc.subcore_barrier()
    pltpu.sync_copy(acc.at[pl.ds(row0, _ROWS_PER_SUB)],
                    out_hbm.at[cid, pl.ds(row0, _ROWS_PER_SUB)])


def _sc_agg(srcs, dsts, ws, h2, zeros2d):
    return pl.kernel(
        _agg_body,
        out_type=jax.ShapeDtypeStruct((_NCORES, _N_PAD, _D), F32),
        mesh=_sc_mesh(),
        scratch_types=[
            pltpu.VMEM((_SBLK, _CHUNK), I32),
            pltpu.VMEM((_SBLK, _CHUNK), I32),
            pltpu.VMEM((_SBLK, _CHUNK), F32),
            pltpu.VMEM((2, _CHUNK, _D), F32),
            pltpu.VMEM_SHARED((_N_PAD, _D), F32),
            pltpu.SemaphoreType.DMA((2,)),
        ],
    )(srcs, dsts, ws, h2, zeros2d)


def _pre1w_body(attr_ref, et_ref, we_ref, be_ref, nt_ref, wl_ref, bl_ref,
                w_ref, tab2_ref):
    t = jnp.dot(et_ref[...], we_ref[...],
                preferred_element_type=F32) + be_ref[...]
    iota = lax.broadcasted_iota(I32, (_EBLK, _D), 1)
    oh = (iota == attr_ref[...]).astype(F32)
    z = jnp.dot(oh, t, preferred_element_type=F32,
                precision=lax.Precision.HIGHEST)
    w_ref[...] = 1.0 / (1.0 + jnp.exp(-z))

    @pl.when(pl.program_id(0) == 0)
    def _():
        tab2_ref[...] = jnp.maximum(
            jnp.dot(nt_ref[...], wl_ref[...], preferred_element_type=F32)
            + bl_ref[...], 0.0)


_EBLK = 8192


def _tc_pre1w(attr2d, et_pad, W_edge, b_edge2, nt_pad, W_lin, b_lin2):
    fix = lambda i: (0, 0)
    return pl.pallas_call(
        _pre1w_body,
        grid=(_E_PAD // _EBLK,),
        in_specs=[
            pl.BlockSpec((_EBLK, 1), lambda i: (i, 0)),
            pl.BlockSpec((_D, _D), fix),
            pl.BlockSpec((_D, 1), fix),
            pl.BlockSpec((1, 1), fix),
            pl.BlockSpec((_D, _D), fix),
            pl.BlockSpec((_D, _D), fix),
            pl.BlockSpec((1, _D), fix),
        ],
        out_specs=(pl.BlockSpec((_EBLK, 1), lambda i: (i, 0)),
                   pl.BlockSpec((_D, _D), fix)),
        out_shape=(jax.ShapeDtypeStruct((_E_PAD, 1), F32),
                   jax.ShapeDtypeStruct((_D, _D), F32)),
    )(attr2d, et_pad, W_edge, b_edge2, nt_pad, W_lin, b_lin2)


_BLK = 512
_NBLK = _N_PAD // _BLK


def _pre2_body(x_ref, dA_ref, dB_ref, tab2_ref, w0_ref,
               h0_ref, dinv_ref, h2_ref):
    deg = dA_ref[...] + dB_ref[...]
    dinv = jnp.where(deg > 0, lax.rsqrt(jnp.maximum(deg, 1e-12)), 0.0)
    dinv2d = jnp.broadcast_to(dinv, (_BLK, _D))
    iota = lax.broadcasted_iota(I32, (_BLK, _D), 1)
    oh = (iota == x_ref[...]).astype(F32)
    h0 = jnp.dot(oh, tab2_ref[...], preferred_element_type=F32, precision=lax.Precision.HIGHEST)
    h0_ref[...] = h0
    dinv_ref[...] = dinv2d
    h2_ref[...] = dinv2d * jnp.dot(h0, w0_ref[...], preferred_element_type=F32)


def _tc_pre2(x2d, degA, degB, tab2, W0):
    blk = lambda i: (i, 0)
    fix = lambda i: (0, 0)
    return pl.pallas_call(
        _pre2_body,
        grid=(_NBLK,),
        in_specs=[
            pl.BlockSpec((_BLK, 1), blk),
            pl.BlockSpec((_BLK, 1), blk),
            pl.BlockSpec((_BLK, 1), blk),
            pl.BlockSpec((_D, _D), fix),
            pl.BlockSpec((_D, _D), fix),
        ],
        out_specs=(pl.BlockSpec((_BLK, _D), blk),
                   pl.BlockSpec((_BLK, _D), blk),
                   pl.BlockSpec((_BLK, _D), blk)),
        out_shape=(jax.ShapeDtypeStruct((_N_PAD, _D), F32),
                   jax.ShapeDtypeStruct((_N_PAD, _D), F32),
                   jax.ShapeDtypeStruct((_N_PAD, _D), F32)),
    )(x2d, degA, degB, tab2, W0)


def _ln_block(s, g_ref, be_ref):
    m = jnp.mean(s, axis=1, keepdims=True)
    d = s - m
    v = jnp.mean(d * d, axis=1, keepdims=True)
    return d / jnp.sqrt(v + 1e-5) * g_ref[...] + be_ref[...]


def _layer_mid_body(p0_ref, p1_ref, dinv_ref, hp_ref, b_ref, g_ref, be_ref,
                    wn_ref, hn_ref, h2_ref):
    dinv = dinv_ref[...]
    agg = dinv * (p0_ref[...] + p1_ref[...]) + b_ref[...]
    s = jnp.maximum(agg, 0.0) + hp_ref[...]
    hn = _ln_block(s, g_ref, be_ref)
    hn_ref[...] = hn
    h2_ref[...] = dinv * jnp.dot(hn, wn_ref[...], preferred_element_type=F32)


def _tc_layer_mid(p0, p1, dinv2d, h_prev, b2, g2, be2, Wn):
    blk = lambda i: (i, 0)
    fix = lambda i: (0, 0)
    return pl.pallas_call(
        _layer_mid_body,
        grid=(_NBLK,),
        in_specs=[
            pl.BlockSpec((_BLK, _D), blk),
            pl.BlockSpec((_BLK, _D), blk),
            pl.BlockSpec((_BLK, _D), blk),
            pl.BlockSpec((_BLK, _D), blk),
            pl.BlockSpec((1, _D), fix),
            pl.BlockSpec((1, _D), fix),
            pl.BlockSpec((1, _D), fix),
            pl.BlockSpec((_D, _D), fix),
        ],
        out_specs=(pl.BlockSpec((_BLK, _D), blk),
                   pl.BlockSpec((_BLK, _D), blk)),
        out_shape=(jax.ShapeDtypeStruct((_N_PAD, _D), F32),
                   jax.ShapeDtypeStruct((_N_PAD, _D), F32)),
    )(p0, p1, dinv2d, h_prev, b2, g2, be2, Wn)


def _layer_last_body(p0_ref, p1_ref, dinv_ref, hp_ref, b_ref, g_ref, be_ref,
                     batch_ref, w1_ref, b1_ref, g1_ref, bb1_ref,
                     w2_ref, b2_ref, g2_ref, bb2_ref, w3_ref, b3_ref,
                     out_ref, pool_ref):
    agg = dinv_ref[...] * (p0_ref[...] + p1_ref[...]) + b_ref[...]
    s = jnp.maximum(agg, 0.0) + hp_ref[...]
    hn = _ln_block(s, g_ref, be_ref)
    gi = lax.broadcasted_iota(I32, (_G, _BLK), 0)
    oh = (gi == batch_ref[...]).astype(F32)
    part = jnp.dot(oh, hn, preferred_element_type=F32,
                   precision=lax.Precision.HIGHEST)

    @pl.when(pl.program_id(0) == 0)
    def _():
        pool_ref[...] = part

    @pl.when(pl.program_id(0) != 0)
    def _():
        pool_ref[...] = pool_ref[...] + part

    @pl.when(pl.program_id(0) == _NBLK - 1)
    def _():
        y = jnp.maximum(jnp.dot(pool_ref[...], w1_ref[...],
                                preferred_element_type=F32) + b1_ref[...], 0.0)
        y = _bn_block(y, g1_ref, bb1_ref)
        y = jnp.maximum(jnp.dot(y, w2_ref[...],
                                preferred_element_type=F32) + b2_ref[...], 0.0)
        y = _bn_block(y, g2_ref, bb2_ref)
        out_ref[...] = jnp.dot(y, w3_ref[...],
                               preferred_element_type=F32) + b3_ref[...]


def _tc_layer_last(p0, p1, dinv2d, h_prev, b2, g2, be2, batch_row,
                   W_fc1, b1, g1, bb1, W_fc2, bb2w, g2b, bb2b, W_fc3, b3):
    blk = lambda i: (i, 0)
    fix = lambda i: (0, 0)
    return pl.pallas_call(
        _layer_last_body,
        grid=(_NBLK,),
        in_specs=[
            pl.BlockSpec((_BLK, _D), blk),
            pl.BlockSpec((_BLK, _D), blk),
            pl.BlockSpec((_BLK, _D), blk),
            pl.BlockSpec((_BLK, _D), blk),
            pl.BlockSpec((1, _D), fix),
            pl.BlockSpec((1, _D), fix),
            pl.BlockSpec((1, _D), fix),
            pl.BlockSpec((1, _BLK), lambda i: (0, i)),
            pl.BlockSpec((_D, 64), fix),
            pl.BlockSpec((1, 64), fix),
            pl.BlockSpec((1, 64), fix),
            pl.BlockSpec((1, 64), fix),
            pl.BlockSpec((64, 32), fix),
            pl.BlockSpec((1, 32), fix),
            pl.BlockSpec((1, 32), fix),
            pl.BlockSpec((1, 32), fix),
            pl.BlockSpec((32, 1), fix),
            pl.BlockSpec((1, 1), fix),
        ],
        out_specs=pl.BlockSpec((_G, 1), fix),
        out_shape=jax.ShapeDtypeStruct((_G, 1), F32),
        scratch_shapes=[pltpu.VMEM((_G, _D), F32)],
    )(p0, p1, dinv2d, h_prev, b2, g2, be2, batch_row,
      W_fc1, b1, g1, bb1, W_fc2, bb2w, g2b, bb2b, W_fc3, b3)


def _bn_block(y, g_ref, b_ref):
    m = jnp.mean(y, axis=0, keepdims=True)
    d = y - m
    v = jnp.mean(d * d, axis=0, keepdims=True)
    return d / jnp.sqrt(v + 1e-5) * g_ref[...] + b_ref[...]


@jax.jit
def _run(x, edge_index, edge_attr, batch, node_table, edge_table,
         W_edge, b_edge, W_lin, b_lin,
         gcn_W0, gcn_b0, ln_g0, ln_b0,
         gcn_W1, gcn_b1, ln_g1, ln_b1,
         gcn_W2, gcn_b2, ln_g2, ln_b2,
         W_fc1, b_fc1, bn1_g, bn1_b,
         W_fc2, b_fc2, bn2_g, bn2_b, W_fc3, b_fc3):
    N = x.shape[0]
    E = edge_index.shape[1]
    npad = _N_PAD - N
    epad = _E_PAD - E

    pad_i = jnp.arange(epad, dtype=I32)
    epack = jnp.concatenate(
        [jnp.stack([edge_index[0].astype(I32),
                    edge_index[1].astype(I32),
                    edge_attr.astype(I32)], axis=0),
         jnp.stack([pad_i % 128, N + (pad_i % npad),
                    jnp.zeros((epad,), I32)], axis=0)], axis=1)
    src_p = epack[0]
    dst_p = epack[1]
    attr_p = epack[2]
    x_p = jnp.concatenate([x.astype(I32), jnp.zeros((npad,), I32)])
    batch_p = jnp.concatenate([batch.astype(I32), jnp.full((npad,), _G, I32)])

    et_pad = jnp.concatenate(
        [edge_table, jnp.zeros((_D - edge_table.shape[0], _D), F32)])
    nt_pad = jnp.concatenate(
        [node_table, jnp.zeros((_D - node_table.shape[0], _D), F32)])

    w2d, tab2 = _tc_pre1w(attr_p.reshape(_E_PAD, 1), et_pad, W_edge,
                          b_edge.reshape(1, 1), nt_pad, W_lin,
                          b_lin.reshape(1, _D))
    src_s = src_p.reshape(_NTOT_SBLK, _SBLK, _CHUNK)
    dst_s = dst_p.reshape(_NTOT_SBLK, _SBLK, _CHUNK)
    w_s = w2d.reshape(_NTOT_SBLK, _SBLK, _CHUNK)
    deg_parts = _sc_prep(dst_s, w_s)
    zeros2d = jnp.zeros((_N_PAD, _D), F32)

    h0, dinv2d, h2 = _tc_pre2(x_p.reshape(_N_PAD, 1),
                              deg_parts[0].reshape(_N_PAD, 1),
                              deg_parts[1].reshape(_N_PAD, 1),
                              tab2, gcn_W0)

    parts = _sc_agg(src_s, dst_s, w_s, h2, zeros2d)
    h1, h2 = _tc_layer_mid(parts[0], parts[1], dinv2d, h0,
                           gcn_b0.reshape(1, _D), ln_g0.reshape(1, _D),
                           ln_b0.reshape(1, _D), gcn_W1)
    parts = _sc_agg(src_s, dst_s, w_s, h2, zeros2d)
    h2r, h2 = _tc_layer_mid(parts[0], parts[1], dinv2d, h1,
                            gcn_b1.reshape(1, _D), ln_g1.reshape(1, _D),
                            ln_b1.reshape(1, _D), gcn_W2)
    parts = _sc_agg(src_s, dst_s, w_s, h2, zeros2d)
    return _tc_layer_last(parts[0], parts[1], dinv2d, h2r,
                          gcn_b2.reshape(1, _D), ln_g2.reshape(1, _D),
                          ln_b2.reshape(1, _D), batch_p.reshape(1, _N_PAD),
                          W_fc1, b_fc1.reshape(1, 64),
                          bn1_g.reshape(1, 64), bn1_b.reshape(1, 64),
                          W_fc2, b_fc2.reshape(1, 32),
                          bn2_g.reshape(1, 32), bn2_b.reshape(1, 32),
                          W_fc3, b_fc3.reshape(1, 1))


def kernel(x, edge_index, edge_attr, batch, size, node_table, edge_table,
           W_edge, b_edge, W_lin, b_lin,
           gcn_W0, gcn_b0, ln_g0, ln_b0,
           gcn_W1, gcn_b1, ln_g1, ln_b1,
           gcn_W2, gcn_b2, ln_g2, ln_b2,
           W_fc1, b_fc1, bn1_g, bn1_b,
           W_fc2, b_fc2, bn2_g, bn2_b, W_fc3, b_fc3):
    del size
    return _run(x, edge_index, edge_attr, batch, node_table, edge_table,
                W_edge, b_edge, W_lin, b_lin,
                gcn_W0, gcn_b0, ln_g0, ln_b0,
                gcn_W1, gcn_b1, ln_g1, ln_b1,
                gcn_W2, gcn_b2, ln_g2, ln_b2,
                W_fc1, b_fc1, bn1_g, bn1_b,
                W_fc2, b_fc2, bn2_g, bn2_b, W_fc3, b_fc3)


# speedup vs baseline: 1.3747x; 1.0288x over previous
import functools

import jax
import jax.numpy as jnp
from jax import lax
from jax.experimental import pallas as pl
from jax.experimental.pallas import tpu as pltpu
from jax.experimental.pallas import tpu_sc as plsc

F32 = jnp.float32
I32 = jnp.int32

_N_PAD = 10240
_E_PAD = 327680
_D = 128
_G = 64
_CHUNK = 128
_NCORES = 2
_NSUB = 16
_NW = _NCORES * _NSUB
_EDGES_PER_W = _E_PAD // _NW
_CHUNKS_PER_W = _EDGES_PER_W // _CHUNK
_ROWS_PER_SUB = _N_PAD // _NSUB


def _sc_mesh():
    return plsc.VectorSubcoreMesh(core_axis_name="c", subcore_axis_name="s",
                                  num_cores=_NCORES, num_subcores=_NSUB)


def _prep_body(dst_hbm, w_hbm, deg_hbm, dst_v, w_v, zero_v, deg_acc):
    cid = lax.axis_index("c")
    sid = lax.axis_index("s")
    wid = cid * _NSUB + sid

    def _z(i, _):
        zero_v[pl.ds(i * 16, 16)] = jnp.zeros((16,), F32)
        return 0
    lax.fori_loop(0, _ROWS_PER_SUB // 16, _z, 0)
    pltpu.sync_copy(zero_v, deg_acc.at[pl.ds(sid * _ROWS_PER_SUB, _ROWS_PER_SUB)])
    plsc.subcore_barrier()

    def _sblk(sb, _):
        si = wid * _NSBLK + sb
        pltpu.sync_copy(dst_hbm.at[si], dst_v)
        pltpu.sync_copy(w_hbm.at[si], w_v)
        for j in range(_SBLK):
            pltpu.sync_copy(w_v.at[j], deg_acc.at[dst_v.at[j]], add=True)
        return 0
    lax.fori_loop(0, _NSBLK, _sblk, 0)

    plsc.subcore_barrier()
    pltpu.sync_copy(deg_acc.at[pl.ds(sid * _ROWS_PER_SUB, _ROWS_PER_SUB)],
                    deg_hbm.at[cid, pl.ds(sid * _ROWS_PER_SUB, _ROWS_PER_SUB)])


def _sc_prep(dsts, ws):
    return pl.kernel(
        _prep_body,
        out_type=jax.ShapeDtypeStruct((_NCORES, _N_PAD), F32),
        mesh=_sc_mesh(),
        scratch_types=[
            pltpu.VMEM((_SBLK, _CHUNK), I32),
            pltpu.VMEM((_SBLK, _CHUNK), F32),
            pltpu.VMEM((_ROWS_PER_SUB,), F32),
            pltpu.VMEM_SHARED((_N_PAD,), F32),
        ],
    )(dsts, ws)


_SBLK = 8
_NSBLK = _CHUNKS_PER_W // _SBLK
_NTOT_SBLK = _E_PAD // (_SBLK * _CHUNK)


def _agg_body(src_hbm, dst_hbm, w_hbm, h2_hbm, zeros_hbm, out_hbm,
              src_v, dst_v, w_v, rows_v, acc, sem):
    cid = lax.axis_index("c")
    sid = lax.axis_index("s")
    wid = cid * _NSUB + sid
    row0 = sid * _ROWS_PER_SUB

    pltpu.sync_copy(zeros_hbm.at[pl.ds(row0, _ROWS_PER_SUB)],
                    acc.at[pl.ds(row0, _ROWS_PER_SUB)])
    plsc.subcore_barrier()

    def _sblk(sb, _):
        si = wid * _NSBLK + sb
        pltpu.sync_copy(src_hbm.at[si], src_v)
        pltpu.sync_copy(dst_hbm.at[si], dst_v)
        pltpu.sync_copy(w_hbm.at[si], w_v)
        pltpu.async_copy(h2_hbm.at[src_v.at[0]], rows_v.at[0], sem.at[0])
        for j in range(_SBLK):
            if j + 1 < _SBLK:
                pltpu.async_copy(h2_hbm.at[src_v.at[j + 1]],
                                 rows_v.at[(j + 1) % 2], sem.at[(j + 1) % 2])
            pltpu.make_async_copy(h2_hbm.at[pl.ds(0, _CHUNK)],
                                  rows_v.at[j % 2], sem.at[j % 2]).wait()

            def _grp(g, _):
                wv = w_v[j, pl.ds(g * 16, 16)]
                for e in range(16):
                    spl = lax.gather(
                        wv, jnp.full((16, 1), e, I32),
                        lax.GatherDimensionNumbers(
                            offset_dims=(), collapsed_slice_dims=(0,),
                            start_index_map=(0,)),
                        slice_sizes=(1,),
                        mode=lax.GatherScatterMode.PROMISE_IN_BOUNDS)
                    r = g * 16 + e
                    for cc in range(8):
                        sl = pl.ds(cc * 16, 16)
                        rows_v[j % 2, r, sl] = rows_v[j % 2, r, sl] * spl
                return 0
            lax.fori_loop(0, _CHUNK // 16, _grp, 0)
            pltpu.sync_copy(rows_v.at[j % 2], acc.at[dst_v.at[j]], add=True)
        return 0
    lax.fori_loop(0, _NSBLK, _sblk, 0)

    plsc.subcore_barrier()
    pltpu.sync_copy(acc.at[pl.ds(row0, _ROWS_PER_SUB)],
                    out_hbm.at[cid, pl.ds(row0, _ROWS_PER_SUB)])


def _sc_agg(srcs, dsts, ws, h2, zeros2d):
    return pl.kernel(
        _agg_body,
        out_type=jax.ShapeDtypeStruct((_NCORES, _N_PAD, _D), F32),
        mesh=_sc_mesh(),
        scratch_types=[
            pltpu.VMEM((_SBLK, _CHUNK), I32),
            pltpu.VMEM((_SBLK, _CHUNK), I32),
            pltpu.VMEM((_SBLK, _CHUNK), F32),
            pltpu.VMEM((2, _CHUNK, _D), F32),
            pltpu.VMEM_SHARED((_N_PAD, _D), F32),
            pltpu.SemaphoreType.DMA((2,)),
        ],
    )(srcs, dsts, ws, h2, zeros2d)


def _pre1w_body(attr_ref, et_ref, we_ref, be_ref, nt_ref, wl_ref, bl_ref,
                w_ref, tab2_ref):
    t = jnp.dot(et_ref[...], we_ref[...],
                preferred_element_type=F32) + be_ref[...]
    iota = lax.broadcasted_iota(I32, (_EBLK, _D), 1)
    oh = (iota == attr_ref[...]).astype(F32)
    z = jnp.dot(oh, t, preferred_element_type=F32,
                precision=lax.Precision.HIGHEST)
    w_ref[...] = 1.0 / (1.0 + jnp.exp(-z))

    @pl.when(pl.program_id(0) == 0)
    def _():
        tab2_ref[...] = jnp.maximum(
            jnp.dot(nt_ref[...], wl_ref[...], preferred_element_type=F32)
            + bl_ref[...], 0.0)


_EBLK = 8192


def _tc_pre1w(attr2d, et_pad, W_edge, b_edge2, nt_pad, W_lin, b_lin2):
    fix = lambda i: (0, 0)
    return pl.pallas_call(
        _pre1w_body,
        grid=(_E_PAD // _EBLK,),
        in_specs=[
            pl.BlockSpec((_EBLK, 1), lambda i: (i, 0)),
            pl.BlockSpec((_D, _D), fix),
            pl.BlockSpec((_D, 1), fix),
            pl.BlockSpec((1, 1), fix),
            pl.BlockSpec((_D, _D), fix),
            pl.BlockSpec((_D, _D), fix),
            pl.BlockSpec((1, _D), fix),
        ],
        out_specs=(pl.BlockSpec((_EBLK, 1), lambda i: (i, 0)),
                   pl.BlockSpec((_D, _D), fix)),
        out_shape=(jax.ShapeDtypeStruct((_E_PAD, 1), F32),
                   jax.ShapeDtypeStruct((_D, _D), F32)),
    )(attr2d, et_pad, W_edge, b_edge2, nt_pad, W_lin, b_lin2)


_BLK = 1024
_NBLK = _N_PAD // _BLK


def _pre2_body(x_ref, dA_ref, dB_ref, tab2_ref, w0_ref,
               h0_ref, dinv_ref, h2_ref):
    deg = dA_ref[...] + dB_ref[...]
    dinv = jnp.where(deg > 0, lax.rsqrt(jnp.maximum(deg, 1e-12)), 0.0)
    dinv2d = jnp.broadcast_to(dinv, (_BLK, _D))
    iota = lax.broadcasted_iota(I32, (_BLK, _D), 1)
    oh = (iota == x_ref[...]).astype(F32)
    h0 = jnp.dot(oh, tab2_ref[...], preferred_element_type=F32, precision=lax.Precision.HIGHEST)
    h0_ref[...] = h0
    dinv_ref[...] = dinv2d
    h2_ref[...] = dinv2d * jnp.dot(h0, w0_ref[...], preferred_element_type=F32)


def _tc_pre2(x2d, degA, degB, tab2, W0):
    blk = lambda i: (i, 0)
    fix = lambda i: (0, 0)
    return pl.pallas_call(
        _pre2_body,
        grid=(_NBLK,),
        in_specs=[
            pl.BlockSpec((_BLK, 1), blk),
            pl.BlockSpec((_BLK, 1), blk),
            pl.BlockSpec((_BLK, 1), blk),
            pl.BlockSpec((_D, _D), fix),
            pl.BlockSpec((_D, _D), fix),
        ],
        out_specs=(pl.BlockSpec((_BLK, _D), blk),
                   pl.BlockSpec((_BLK, _D), blk),
                   pl.BlockSpec((_BLK, _D), blk)),
        out_shape=(jax.ShapeDtypeStruct((_N_PAD, _D), F32),
                   jax.ShapeDtypeStruct((_N_PAD, _D), F32),
                   jax.ShapeDtypeStruct((_N_PAD, _D), F32)),
    )(x2d, degA, degB, tab2, W0)


def _ln_block(s, g_ref, be_ref):
    m = jnp.mean(s, axis=1, keepdims=True)
    d = s - m
    v = jnp.mean(d * d, axis=1, keepdims=True)
    return d / jnp.sqrt(v + 1e-5) * g_ref[...] + be_ref[...]


def _layer_mid_body(p0_ref, p1_ref, dinv_ref, hp_ref, b_ref, g_ref, be_ref,
                    wn_ref, hn_ref, h2_ref):
    dinv = dinv_ref[...]
    agg = dinv * (p0_ref[...] + p1_ref[...]) + b_ref[...]
    s = jnp.maximum(agg, 0.0) + hp_ref[...]
    hn = _ln_block(s, g_ref, be_ref)
    hn_ref[...] = hn
    h2_ref[...] = dinv * jnp.dot(hn, wn_ref[...], preferred_element_type=F32)


def _tc_layer_mid(p0, p1, dinv2d, h_prev, b2, g2, be2, Wn):
    blk = lambda i: (i, 0)
    fix = lambda i: (0, 0)
    return pl.pallas_call(
        _layer_mid_body,
        grid=(_NBLK,),
        in_specs=[
            pl.BlockSpec((_BLK, _D), blk),
            pl.BlockSpec((_BLK, _D), blk),
            pl.BlockSpec((_BLK, _D), blk),
            pl.BlockSpec((_BLK, _D), blk),
            pl.BlockSpec((1, _D), fix),
            pl.BlockSpec((1, _D), fix),
            pl.BlockSpec((1, _D), fix),
            pl.BlockSpec((_D, _D), fix),
        ],
        out_specs=(pl.BlockSpec((_BLK, _D), blk),
                   pl.BlockSpec((_BLK, _D), blk)),
        out_shape=(jax.ShapeDtypeStruct((_N_PAD, _D), F32),
                   jax.ShapeDtypeStruct((_N_PAD, _D), F32)),
    )(p0, p1, dinv2d, h_prev, b2, g2, be2, Wn)


def _layer_last_body(p0_ref, p1_ref, dinv_ref, hp_ref, b_ref, g_ref, be_ref,
                     batch_ref, w1_ref, b1_ref, g1_ref, bb1_ref,
                     w2_ref, b2_ref, g2_ref, bb2_ref, w3_ref, b3_ref,
                     out_ref, pool_ref):
    agg = dinv_ref[...] * (p0_ref[...] + p1_ref[...]) + b_ref[...]
    s = jnp.maximum(agg, 0.0) + hp_ref[...]
    hn = _ln_block(s, g_ref, be_ref)
    gi = lax.broadcasted_iota(I32, (_G, _BLK), 0)
    oh = (gi == batch_ref[...]).astype(F32)
    part = jnp.dot(oh, hn, preferred_element_type=F32,
                   precision=lax.Precision.HIGHEST)

    @pl.when(pl.program_id(0) == 0)
    def _():
        pool_ref[...] = part

    @pl.when(pl.program_id(0) != 0)
    def _():
        pool_ref[...] = pool_ref[...] + part

    @pl.when(pl.program_id(0) == _NBLK - 1)
    def _():
        y = jnp.maximum(jnp.dot(pool_ref[...], w1_ref[...],
                                preferred_element_type=F32) + b1_ref[...], 0.0)
        y = _bn_block(y, g1_ref, bb1_ref)
        y = jnp.maximum(jnp.dot(y, w2_ref[...],
                                preferred_element_type=F32) + b2_ref[...], 0.0)
        y = _bn_block(y, g2_ref, bb2_ref)
        out_ref[...] = jnp.dot(y, w3_ref[...],
                               preferred_element_type=F32) + b3_ref[...]


def _tc_layer_last(p0, p1, dinv2d, h_prev, b2, g2, be2, batch_row,
                   W_fc1, b1, g1, bb1, W_fc2, bb2w, g2b, bb2b, W_fc3, b3):
    blk = lambda i: (i, 0)
    fix = lambda i: (0, 0)
    return pl.pallas_call(
        _layer_last_body,
        grid=(_NBLK,),
        in_specs=[
            pl.BlockSpec((_BLK, _D), blk),
            pl.BlockSpec((_BLK, _D), blk),
            pl.BlockSpec((_BLK, _D), blk),
            pl.BlockSpec((_BLK, _D), blk),
            pl.BlockSpec((1, _D), fix),
            pl.BlockSpec((1, _D), fix),
            pl.BlockSpec((1, _D), fix),
            pl.BlockSpec((1, _BLK), lambda i: (0, i)),
            pl.BlockSpec((_D, 64), fix),
            pl.BlockSpec((1, 64), fix),
            pl.BlockSpec((1, 64), fix),
            pl.BlockSpec((1, 64), fix),
            pl.BlockSpec((64, 32), fix),
            pl.BlockSpec((1, 32), fix),
            pl.BlockSpec((1, 32), fix),
            pl.BlockSpec((1, 32), fix),
            pl.BlockSpec((32, 1), fix),
            pl.BlockSpec((1, 1), fix),
        ],
        out_specs=pl.BlockSpec((_G, 1), fix),
        out_shape=jax.ShapeDtypeStruct((_G, 1), F32),
        scratch_shapes=[pltpu.VMEM((_G, _D), F32)],
    )(p0, p1, dinv2d, h_prev, b2, g2, be2, batch_row,
      W_fc1, b1, g1, bb1, W_fc2, bb2w, g2b, bb2b, W_fc3, b3)


def _bn_block(y, g_ref, b_ref):
    m = jnp.mean(y, axis=0, keepdims=True)
    d = y - m
    v = jnp.mean(d * d, axis=0, keepdims=True)
    return d / jnp.sqrt(v + 1e-5) * g_ref[...] + b_ref[...]


@jax.jit
def _run(x, edge_index, edge_attr, batch, node_table, edge_table,
         W_edge, b_edge, W_lin, b_lin,
         gcn_W0, gcn_b0, ln_g0, ln_b0,
         gcn_W1, gcn_b1, ln_g1, ln_b1,
         gcn_W2, gcn_b2, ln_g2, ln_b2,
         W_fc1, b_fc1, bn1_g, bn1_b,
         W_fc2, b_fc2, bn2_g, bn2_b, W_fc3, b_fc3):
    N = x.shape[0]
    E = edge_index.shape[1]
    npad = _N_PAD - N
    epad = _E_PAD - E

    pad_i = jnp.arange(epad, dtype=I32)
    epack = jnp.concatenate(
        [jnp.stack([edge_index[0].astype(I32),
                    edge_index[1].astype(I32),
                    edge_attr.astype(I32)], axis=0),
         jnp.stack([pad_i % 128, N + (pad_i % npad),
                    jnp.zeros((epad,), I32)], axis=0)], axis=1)
    src_p = epack[0]
    dst_p = epack[1]
    attr_p = epack[2]
    x_p = jnp.concatenate([x.astype(I32), jnp.zeros((npad,), I32)])
    batch_p = jnp.concatenate([batch.astype(I32), jnp.full((npad,), _G, I32)])

    et_pad = jnp.concatenate(
        [edge_table, jnp.zeros((_D - edge_table.shape[0], _D), F32)])
    nt_pad = jnp.concatenate(
        [node_table, jnp.zeros((_D - node_table.shape[0], _D), F32)])

    w2d, tab2 = _tc_pre1w(attr_p.reshape(_E_PAD, 1), et_pad, W_edge,
                          b_edge.reshape(1, 1), nt_pad, W_lin,
                          b_lin.reshape(1, _D))
    src_s = src_p.reshape(_NTOT_SBLK, _SBLK, _CHUNK)
    dst_s = dst_p.reshape(_NTOT_SBLK, _SBLK, _CHUNK)
    w_s = w2d.reshape(_NTOT_SBLK, _SBLK, _CHUNK)
    deg_parts = _sc_prep(dst_s, w_s)
    zeros2d = jnp.zeros((_N_PAD, _D), F32)

    h0, dinv2d, h2 = _tc_pre2(x_p.reshape(_N_PAD, 1),
                              deg_parts[0].reshape(_N_PAD, 1),
                              deg_parts[1].reshape(_N_PAD, 1),
                              tab2, gcn_W0)

    parts = _sc_agg(src_s, dst_s, w_s, h2, zeros2d)
    h1, h2 = _tc_layer_mid(parts[0], parts[1], dinv2d, h0,
                           gcn_b0.reshape(1, _D), ln_g0.reshape(1, _D),
                           ln_b0.reshape(1, _D), gcn_W1)
    parts = _sc_agg(src_s, dst_s, w_s, h2, zeros2d)
    h2r, h2 = _tc_layer_mid(parts[0], parts[1], dinv2d, h1,
                            gcn_b1.reshape(1, _D), ln_g1.reshape(1, _D),
                            ln_b1.reshape(1, _D), gcn_W2)
    parts = _sc_agg(src_s, dst_s, w_s, h2, zeros2d)
    return _tc_layer_last(parts[0], parts[1], dinv2d, h2r,
                          gcn_b2.reshape(1, _D), ln_g2.reshape(1, _D),
                          ln_b2.reshape(1, _D), batch_p.reshape(1, _N_PAD),
                          W_fc1, b_fc1.reshape(1, 64),
                          bn1_g.reshape(1, 64), bn1_b.reshape(1, 64),
                          W_fc2, b_fc2.reshape(1, 32),
                          bn2_g.reshape(1, 32), bn2_b.reshape(1, 32),
                          W_fc3, b_fc3.reshape(1, 1))


def kernel(x, edge_index, edge_attr, batch, size, node_table, edge_table,
           W_edge, b_edge, W_lin, b_lin,
           gcn_W0, gcn_b0, ln_g0, ln_b0,
           gcn_W1, gcn_b1, ln_g1, ln_b1,
           gcn_W2, gcn_b2, ln_g2, ln_b2,
           W_fc1, b_fc1, bn1_g, bn1_b,
           W_fc2, b_fc2, bn2_g, bn2_b, W_fc3, b_fc3):
    del size
    return _run(x, edge_index, edge_attr, batch, node_table, edge_table,
                W_edge, b_edge, W_lin, b_lin,
                gcn_W0, gcn_b0, ln_g0, ln_b0,
                gcn_W1, gcn_b1, ln_g1, ln_b1,
                gcn_W2, gcn_b2, ln_g2, ln_b2,
                W_fc1, b_fc1, bn1_g, bn1_b,
                W_fc2, b_fc2, bn2_g, bn2_b, W_fc3, b_fc3)
